# Initial kernel scaffold; baseline (speedup 1.0000x reference)
import jax
import jax.numpy as jnp
from jax.experimental import pallas as pl


def kernel(x, edge_index, edge_label_index, W1l, W1r, b1, W2l, W2r, b2):
    raise NotImplementedError("write your pallas kernel here")


# speedup vs baseline: 8.8063x; 8.8063x over previous
import functools

import jax
import jax.numpy as jnp
from jax import lax
from jax.experimental import pallas as pl
from jax.experimental.pallas import tpu as pltpu
from jax.experimental.pallas import tpu_sc as plsc

N_NODES = 10000
IN_CH = 256
HID_CH = 512
OUT_CH = 256
N_EDGES = 160000
N_LABEL = 20000

NC = 2
NT = 16
HALF = 128

KB = 128
NB = 80
E_PAD = NT * NB * KB
ROWS_PT = 632
N_PAD_ROWS = NT * ROWS_PT

DK = 64
DNB = 10
NW = NC * NT
L_PAD = NW * DNB * DK

_f32 = jnp.float32
_i32 = jnp.int32


NCH = 8
NCHUNK = NB // NCH


def _agg_body(tbl, srcs2, dsts, agg_out, cnt_out,
              agg_acc, cnt_acc,
              src_ch0, src_ch1, dst_ch0, dst_ch1, msg0, msg1, ones_v,
              gsem0, gsem1):
    c = lax.axis_index("c")
    s = lax.axis_index("s")
    msg = (msg0, msg1)
    gsem = (gsem0, gsem1)
    src_ch = (src_ch0, src_ch1)
    dst_ch = (dst_ch0, dst_ch1)

    zero16 = jnp.zeros((16,), _f32)

    @pl.loop(0, KB)
    def _zmsg(r):
        for j in range(HALF // 16):
            msg0[r, pl.ds(j * 16, 16)] = zero16
        ones_v[r, pl.ds(0, 16)] = zero16

    for t in range(ROWS_PT // KB):
        pltpu.sync_copy(msg0, agg_acc.at[pl.ds(s * ROWS_PT + t * KB, KB)])
    _rem = ROWS_PT % KB
    pltpu.sync_copy(msg0.at[pl.ds(0, _rem)],
                    agg_acc.at[pl.ds(s * ROWS_PT + (ROWS_PT // KB) * KB, _rem)])

    @pl.when(c == 0)
    def _():
        for t in range(ROWS_PT // KB):
            pltpu.sync_copy(ones_v, cnt_acc.at[pl.ds(s * ROWS_PT + t * KB, KB)])
        pltpu.sync_copy(ones_v.at[pl.ds(0, _rem)],
                        cnt_acc.at[pl.ds(s * ROWS_PT + (ROWS_PT // KB) * KB, _rem)])

    one16 = jnp.ones((16,), _f32)

    @pl.loop(0, KB)
    def _ones(r):
        ones_v[r, pl.ds(0, 16)] = one16

    pltpu.sync_copy(srcs2.at[c, s, pl.ds(0, NCH)], src_ch0)
    pltpu.sync_copy(dsts.at[s, pl.ds(0, NCH)], dst_ch0)

    plsc.subcore_barrier()

    pltpu.async_copy(tbl.at[src_ch0.at[0]], msg0, gsem0)
    pltpu.async_copy(tbl.at[src_ch0.at[1]], msg1, gsem1)

    @pl.loop(0, NCHUNK // 2)
    def _pipe(kp):
        for kb in range(2):
            k = kp * 2 + kb

            @pl.when(k < NCHUNK - 1)
            def _():
                pltpu.sync_copy(srcs2.at[c, s, pl.ds((k + 1) * NCH, NCH)],
                                src_ch[1 - kb])
                pltpu.sync_copy(dsts.at[s, pl.ds((k + 1) * NCH, NCH)],
                                dst_ch[1 - kb])

            for b8 in range(NCH):
                i = k * NCH + b8
                b = b8 % 2
                pltpu.make_async_copy(tbl.at[src_ch[kb].at[b8]],
                                      msg[b], gsem[b]).wait()
                pltpu.sync_copy(msg[b], agg_acc.at[dst_ch[kb].at[b8]], add=True)

                @pl.when(c == 0)
                def _():
                    pltpu.sync_copy(ones_v, cnt_acc.at[dst_ch[kb].at[b8]],
                                    add=True)

                @pl.when(i < NB - 2)
                def _():
                    if b8 < NCH - 2:
                        nxt = src_ch[kb].at[b8 + 2]
                    else:
                        nxt = src_ch[1 - kb].at[b8 + 2 - NCH]
                    pltpu.async_copy(tbl.at[nxt], msg[b], gsem[b])

    plsc.subcore_barrier()

    for t in range(ROWS_PT // KB + 1):
        rows = KB if t < ROWS_PT // KB else ROWS_PT % KB
        r0 = s * ROWS_PT + t * KB
        pltpu.sync_copy(agg_acc.at[pl.ds(r0, rows)], msg0.at[pl.ds(0, rows)])
        pltpu.sync_copy(msg0.at[pl.ds(0, rows)],
                        agg_out.at[c, pl.ds(r0, rows)])

    @pl.when(c == 0)
    def _():
        for t in range(ROWS_PT // KB + 1):
            rows = KB if t < ROWS_PT // KB else ROWS_PT % KB
            r0 = s * ROWS_PT + t * KB
            pltpu.sync_copy(cnt_acc.at[pl.ds(r0, rows)],
                            ones_v.at[pl.ds(0, rows)])
            pltpu.sync_copy(ones_v.at[pl.ds(0, rows)],
                            cnt_out.at[pl.ds(r0, rows)])


def _make_agg_kernel():
    return pl.kernel(
        _agg_body,
        out_type=(
            jax.ShapeDtypeStruct((NC, N_PAD_ROWS, HALF), _f32),
            jax.ShapeDtypeStruct((N_PAD_ROWS, 16), _f32),
        ),
        mesh=plsc.VectorSubcoreMesh(core_axis_name="c", subcore_axis_name="s"),
        compiler_params=pltpu.CompilerParams(use_tc_tiling_on_sc=False),
        scratch_types=[
            pltpu.VMEM_SHARED((N_PAD_ROWS, HALF), _f32),
            pltpu.VMEM_SHARED((N_PAD_ROWS, 16), _f32),
            pltpu.VMEM((NCH, KB), _i32),
            pltpu.VMEM((NCH, KB), _i32),
            pltpu.VMEM((NCH, KB), _i32),
            pltpu.VMEM((NCH, KB), _i32),
            pltpu.VMEM((KB, HALF), _f32),
            pltpu.VMEM((KB, HALF), _f32),
            pltpu.VMEM((KB, 16), _f32),
            pltpu.SemaphoreType.DMA,
            pltpu.SemaphoreType.DMA,
        ],
    )


def _decode_body(z, sls, dls, scores,
                 sl_all, dl_all, sr0, sr1, dr0, dr1, out_v,
                 ss0, ss1, sd0, sd1):
    c = lax.axis_index("c")
    s = lax.axis_index("s")
    w = c * NT + s
    sr = (sr0, sr1)
    dr = (dr0, dr1)
    ssem = (ss0, ss1)
    dsem = (sd0, sd1)

    pltpu.sync_copy(sls.at[w], sl_all)
    pltpu.sync_copy(dls.at[w], dl_all)

    pltpu.async_copy(z.at[sl_all.at[pl.ds(0, DK)]], sr0, ss0)
    pltpu.async_copy(z.at[dl_all.at[pl.ds(0, DK)]], dr0, sd0)
    pltpu.async_copy(z.at[sl_all.at[pl.ds(DK, DK)]], sr1, ss1)
    pltpu.async_copy(z.at[dl_all.at[pl.ds(DK, DK)]], dr1, sd1)

    lane = lax.iota(_i32, 16)

    @pl.loop(0, DNB // 2)
    def _pipe(io):
        for b in range(2):
            i = io * 2 + b
            pltpu.make_async_copy(z.at[sl_all.at[pl.ds(0, DK)]], sr[b], ssem[b]).wait()
            pltpu.make_async_copy(z.at[dl_all.at[pl.ds(0, DK)]], dr[b], dsem[b]).wait()

            @pl.loop(0, DK // 16)
            def _grp(g):
                vec = jnp.zeros((16,), _f32)
                for j in range(16):
                    e = g * 16 + j
                    acc = sr[b][e, pl.ds(0, 16)] * dr[b][e, pl.ds(0, 16)]
                    for f in range(1, OUT_CH // 16):
                        acc = acc + (sr[b][e, pl.ds(f * 16, 16)]
                                     * dr[b][e, pl.ds(f * 16, 16)])
                    vec = jnp.where(lane == j, jnp.sum(acc), vec)
                out_v[pl.ds(g * 16, 16)] = vec

            @pl.when(io < DNB // 2 - 1)
            def _():
                pltpu.async_copy(z.at[sl_all.at[pl.ds((i + 2) * DK, DK)]],
                                 sr[b], ssem[b])
                pltpu.async_copy(z.at[dl_all.at[pl.ds((i + 2) * DK, DK)]],
                                 dr[b], dsem[b])

            pltpu.sync_copy(out_v, scores.at[pl.ds(w * DNB * DK + i * DK, DK)])


_decode_kernel = pl.kernel(
    _decode_body,
    out_type=jax.ShapeDtypeStruct((L_PAD,), _f32),
    mesh=plsc.VectorSubcoreMesh(core_axis_name="c", subcore_axis_name="s"),
    compiler_params=pltpu.CompilerParams(use_tc_tiling_on_sc=False,
                                         needs_layout_passes=False),
    scratch_types=[
        pltpu.VMEM((DNB * DK,), _i32),
        pltpu.VMEM((DNB * DK,), _i32),
        pltpu.VMEM((DK, OUT_CH), _f32),
        pltpu.VMEM((DK, OUT_CH), _f32),
        pltpu.VMEM((DK, OUT_CH), _f32),
        pltpu.VMEM((DK, OUT_CH), _f32),
        pltpu.VMEM((DK,), _f32),
        pltpu.SemaphoreType.DMA,
        pltpu.SemaphoreType.DMA,
        pltpu.SemaphoreType.DMA,
        pltpu.SemaphoreType.DMA,
    ],
)


_R = 1000


def _encode_tc_body(a0, a1, cnt, xb, w1l0, w1l1, w1r, b1, w2l, w2r, b2,
                    p_out, r_out):
    inv = 1.0 / jnp.maximum(cnt[:, 0:1], 1.0)
    h = (jnp.dot(a0[...] * inv, w1l0[...], preferred_element_type=_f32)
         + jnp.dot(a1[...] * inv, w1l1[...], preferred_element_type=_f32)
         + jnp.dot(xb[...], w1r[...], preferred_element_type=_f32)
         + b1[...])
    h = jnp.maximum(h, 0.0)
    p_out[...] = jnp.dot(h, w2l[...], preferred_element_type=_f32)
    r_out[...] = jnp.dot(h, w2r[...], preferred_element_type=_f32) + b2[...]


def _combine_tc_body(a0, a1, cnt, r, z_out):
    inv = 1.0 / jnp.maximum(cnt[:, 0:1], 1.0)
    m = jnp.concatenate([a0[...], a1[...]], axis=-1)
    z_out[...] = m * inv + r[...]


def _full(shape):
    return pl.BlockSpec(shape, lambda i: (0,) * len(shape))


def _rows(shape):
    return pl.BlockSpec(shape, lambda i: (i,) + (0,) * (len(shape) - 1))


_encode_tc = pl.pallas_call(
    _encode_tc_body,
    grid=(N_NODES // _R,),
    in_specs=[
        _rows((_R, HALF)), _rows((_R, HALF)), _rows((_R, 16)),
        _rows((_R, IN_CH)),
        _full((HALF, HID_CH)), _full((HALF, HID_CH)), _full((IN_CH, HID_CH)),
        _full((1, HID_CH)),
        _full((HID_CH, OUT_CH)), _full((HID_CH, OUT_CH)), _full((1, OUT_CH)),
    ],
    out_specs=[_rows((_R, OUT_CH)), _rows((_R, OUT_CH))],
    out_shape=[
        jax.ShapeDtypeStruct((N_NODES, OUT_CH), _f32),
        jax.ShapeDtypeStruct((N_NODES, OUT_CH), _f32),
    ],
)

_combine_tc = pl.pallas_call(
    _combine_tc_body,
    grid=(N_NODES // _R,),
    in_specs=[
        _rows((_R, HALF)), _rows((_R, HALF)), _rows((_R, 16)),
        _rows((_R, OUT_CH)),
    ],
    out_specs=_rows((_R, OUT_CH)),
    out_shape=jax.ShapeDtypeStruct((N_NODES, OUT_CH), _f32),
)


def kernel(x, edge_index, edge_label_index, W1l, W1r, b1, W2l, W2r, b2):
    src = edge_index[0].astype(_i32)
    dst = edge_index[1].astype(_i32)
    sl = edge_label_index[0].astype(_i32)
    dl = edge_label_index[1].astype(_i32)

    ep = E_PAD - N_EDGES
    pad_src = (jnp.arange(ep, dtype=_i32) * 37) % N_NODES
    pad_dst = N_NODES + (jnp.arange(ep, dtype=_i32) % (N_PAD_ROWS - N_NODES))
    src_p = jnp.concatenate([src, pad_src]).reshape(NT, NB, KB)
    srcs2 = jnp.stack([2 * src_p, 2 * src_p + 1])
    dsts = jnp.concatenate([dst, pad_dst]).reshape(NT, NB, KB)

    agg_k = _make_agg_kernel()

    agg1, cnt = agg_k(x.reshape(2 * N_NODES, HALF), srcs2, dsts)
    p, r = _encode_tc(agg1[0], agg1[1], cnt, x,
                      W1l[:HALF], W1l[HALF:], W1r, b1.reshape(1, HID_CH),
                      W2l, W2r, b2.reshape(1, OUT_CH))

    agg2, _ = agg_k(p.reshape(2 * N_NODES, HALF), srcs2, dsts)
    z = _combine_tc(agg2[0], agg2[1], cnt, r)

    lp = L_PAD - N_LABEL
    pad_l = (jnp.arange(lp, dtype=_i32) * 41) % N_NODES
    sls = jnp.concatenate([sl, pad_l]).reshape(NW, DNB * DK)
    dls = jnp.concatenate([dl, pad_l]).reshape(NW, DNB * DK)
    scores = _decode_kernel(z, sls, dls)
    return scores[:N_LABEL]


# speedup vs baseline: 9.4045x; 1.0679x over previous
import functools

import jax
import jax.numpy as jnp
from jax import lax
from jax.experimental import pallas as pl
from jax.experimental.pallas import tpu as pltpu
from jax.experimental.pallas import tpu_sc as plsc

N_NODES = 10000
IN_CH = 256
HID_CH = 512
OUT_CH = 256
N_EDGES = 160000
N_LABEL = 20000

NC = 2
NT = 16
HALF = 128

KB = 128
NB = 80
E_PAD = NT * NB * KB
ROWS_PT = 632
N_PAD_ROWS = NT * ROWS_PT

DK = 64
DNB = 10
NW = NC * NT
L_PAD = NW * DNB * DK

_f32 = jnp.float32
_i32 = jnp.int32


NCH = 8
NCHUNK = NB // NCH


def _agg_body(with_cnt, *refs):
    if with_cnt:
        (tbl, srcs2, dsts, agg_out, cnt_out,
         agg_acc, cnt_acc,
         src_ch0, src_ch1, dst_ch0, dst_ch1, msg0, msg1, ones_v,
         gsem0, gsem1, chsem, csem, zsem) = refs
    else:
        (tbl, srcs2, dsts, agg_out,
         agg_acc,
         src_ch0, src_ch1, dst_ch0, dst_ch1, msg0, msg1,
         gsem0, gsem1, chsem, zsem) = refs
    c = lax.axis_index("c")
    s = lax.axis_index("s")
    msg = (msg0, msg1)
    gsem = (gsem0, gsem1)
    src_ch = (src_ch0, src_ch1)
    dst_ch = (dst_ch0, dst_ch1)

    zero16 = jnp.zeros((16,), _f32)

    @pl.loop(0, KB)
    def _zmsg(r):
        for j in range(HALF // 16):
            msg0[r, pl.ds(j * 16, 16)] = zero16
        if with_cnt:
            ones_v[r, pl.ds(0, 16)] = zero16

    _nfull = ROWS_PT // KB
    _rem = ROWS_PT % KB
    for t in range(_nfull):
        pltpu.async_copy(msg0, agg_acc.at[pl.ds(s * ROWS_PT + t * KB, KB)],
                         zsem)
    pltpu.async_copy(msg0.at[pl.ds(0, _rem)],
                     agg_acc.at[pl.ds(s * ROWS_PT + _nfull * KB, _rem)], zsem)

    if with_cnt:
        @pl.when(c == 0)
        def _():
            for t in range(_nfull):
                pltpu.async_copy(
                    ones_v, cnt_acc.at[pl.ds(s * ROWS_PT + t * KB, KB)], zsem)
            pltpu.async_copy(
                ones_v.at[pl.ds(0, _rem)],
                cnt_acc.at[pl.ds(s * ROWS_PT + _nfull * KB, _rem)], zsem)

    pltpu.async_copy(srcs2.at[c, s, pl.ds(0, NCH)], src_ch0, chsem)
    pltpu.async_copy(dsts.at[s, pl.ds(0, NCH)], dst_ch0, chsem)

    for t in range(_nfull):
        pltpu.make_async_copy(
            msg0, agg_acc.at[pl.ds(s * ROWS_PT, KB)], zsem).wait()
    pltpu.make_async_copy(
        msg0.at[pl.ds(0, _rem)], agg_acc.at[pl.ds(s * ROWS_PT, _rem)],
        zsem).wait()
    if with_cnt:
        @pl.when(c == 0)
        def _():
            for t in range(_nfull):
                pltpu.make_async_copy(
                    ones_v, cnt_acc.at[pl.ds(s * ROWS_PT, KB)], zsem).wait()
            pltpu.make_async_copy(
                ones_v.at[pl.ds(0, _rem)],
                cnt_acc.at[pl.ds(s * ROWS_PT, _rem)], zsem).wait()

        one16 = jnp.ones((16,), _f32)

        @pl.loop(0, KB)
        def _ones(r):
            ones_v[r, pl.ds(0, 16)] = one16

    pltpu.make_async_copy(srcs2.at[c, s, pl.ds(0, NCH)], src_ch0, chsem).wait()
    pltpu.make_async_copy(dsts.at[s, pl.ds(0, NCH)], dst_ch0, chsem).wait()

    plsc.subcore_barrier()

    pltpu.async_copy(tbl.at[src_ch0.at[0]], msg0, gsem0)
    pltpu.async_copy(tbl.at[src_ch0.at[1]], msg1, gsem1)

    @pl.loop(0, NCHUNK // 2)
    def _pipe(kp):
        for kb in range(2):
            k = kp * 2 + kb

            @pl.when(k < NCHUNK - 1)
            def _():
                pltpu.async_copy(srcs2.at[c, s, pl.ds((k + 1) * NCH, NCH)],
                                 src_ch[1 - kb], chsem)
                pltpu.async_copy(dsts.at[s, pl.ds((k + 1) * NCH, NCH)],
                                 dst_ch[1 - kb], chsem)

            for b8 in range(NCH):
                i = k * NCH + b8
                b = b8 % 2
                if b8 == NCH - 2:
                    @pl.when(k < NCHUNK - 1)
                    def _():
                        pltpu.make_async_copy(
                            srcs2.at[c, s, pl.ds(0, NCH)], src_ch[1 - kb],
                            chsem).wait()
                        pltpu.make_async_copy(
                            dsts.at[s, pl.ds(0, NCH)], dst_ch[1 - kb],
                            chsem).wait()
                pltpu.make_async_copy(tbl.at[src_ch[kb].at[b8]],
                                      msg[b], gsem[b]).wait()
                pltpu.sync_copy(msg[b], agg_acc.at[dst_ch[kb].at[b8]], add=True)

                if with_cnt:
                    @pl.when(c == 0)
                    def _():
                        pltpu.async_copy(ones_v,
                                         cnt_acc.at[dst_ch[kb].at[b8]],
                                         csem, add=True)

                @pl.when(i < NB - 2)
                def _():
                    if b8 < NCH - 2:
                        nxt = src_ch[kb].at[b8 + 2]
                    else:
                        nxt = src_ch[1 - kb].at[b8 + 2 - NCH]
                    pltpu.async_copy(tbl.at[nxt], msg[b], gsem[b])

    if with_cnt:
        @pl.when(c == 0)
        def _():
            @pl.loop(0, NB)
            def _drain(_):
                pltpu.make_async_copy(
                    ones_v, cnt_acc.at[dst_ch0.at[0]], csem).wait()

    plsc.subcore_barrier()

    for t in range(_nfull + 1):
        rows = KB if t < _nfull else _rem
        r0 = s * ROWS_PT + t * KB
        pltpu.sync_copy(agg_acc.at[pl.ds(r0, rows)], msg0.at[pl.ds(0, rows)])
        pltpu.sync_copy(msg0.at[pl.ds(0, rows)],
                        agg_out.at[c, pl.ds(r0, rows)])

    if with_cnt:
        @pl.when(c == 0)
        def _():
            for t in range(_nfull + 1):
                rows = KB if t < _nfull else _rem
                r0 = s * ROWS_PT + t * KB
                pltpu.sync_copy(cnt_acc.at[pl.ds(r0, rows)],
                                ones_v.at[pl.ds(0, rows)])
                pltpu.sync_copy(ones_v.at[pl.ds(0, rows)],
                                cnt_out.at[pl.ds(r0, rows)])


def _make_agg_kernel(with_cnt):
    if with_cnt:
        out_type = (
            jax.ShapeDtypeStruct((NC, N_PAD_ROWS, HALF), _f32),
            jax.ShapeDtypeStruct((N_PAD_ROWS, 16), _f32),
        )
    else:
        out_type = jax.ShapeDtypeStruct((NC, N_PAD_ROWS, HALF), _f32)
    scratch = [pltpu.VMEM_SHARED((N_PAD_ROWS, HALF), _f32)]
    if with_cnt:
        scratch.append(pltpu.VMEM_SHARED((N_PAD_ROWS, 16), _f32))
    scratch += [
        pltpu.VMEM((NCH, KB), _i32),
        pltpu.VMEM((NCH, KB), _i32),
        pltpu.VMEM((NCH, KB), _i32),
        pltpu.VMEM((NCH, KB), _i32),
        pltpu.VMEM((KB, HALF), _f32),
        pltpu.VMEM((KB, HALF), _f32),
    ]
    if with_cnt:
        scratch.append(pltpu.VMEM((KB, 16), _f32))
    scratch += [pltpu.SemaphoreType.DMA, pltpu.SemaphoreType.DMA,
                pltpu.SemaphoreType.DMA]
    if with_cnt:
        scratch.append(pltpu.SemaphoreType.DMA)
    scratch.append(pltpu.SemaphoreType.DMA)
    return pl.kernel(
        functools.partial(_agg_body, with_cnt),
        out_type=out_type,
        mesh=plsc.VectorSubcoreMesh(core_axis_name="c", subcore_axis_name="s"),
        compiler_params=pltpu.CompilerParams(use_tc_tiling_on_sc=False),
        scratch_types=scratch,
    )


def _decode_body(z, sls, dls, scores,
                 sl_all, dl_all, sr0, sr1, dr0, dr1, out_v,
                 ss0, ss1, sd0, sd1):
    c = lax.axis_index("c")
    s = lax.axis_index("s")
    w = c * NT + s
    sr = (sr0, sr1)
    dr = (dr0, dr1)
    ssem = (ss0, ss1)
    dsem = (sd0, sd1)

    pltpu.sync_copy(sls.at[w], sl_all)
    pltpu.sync_copy(dls.at[w], dl_all)

    pltpu.async_copy(z.at[sl_all.at[pl.ds(0, DK)]], sr0, ss0)
    pltpu.async_copy(z.at[dl_all.at[pl.ds(0, DK)]], dr0, sd0)
    pltpu.async_copy(z.at[sl_all.at[pl.ds(DK, DK)]], sr1, ss1)
    pltpu.async_copy(z.at[dl_all.at[pl.ds(DK, DK)]], dr1, sd1)

    lane = lax.iota(_i32, 16)

    @pl.loop(0, DNB // 2)
    def _pipe(io):
        for b in range(2):
            i = io * 2 + b
            pltpu.make_async_copy(z.at[sl_all.at[pl.ds(0, DK)]], sr[b], ssem[b]).wait()
            pltpu.make_async_copy(z.at[dl_all.at[pl.ds(0, DK)]], dr[b], dsem[b]).wait()

            @pl.loop(0, DK // 16)
            def _grp(g):
                vec = jnp.zeros((16,), _f32)
                for j in range(16):
                    e = g * 16 + j
                    acc = sr[b][e, pl.ds(0, 16)] * dr[b][e, pl.ds(0, 16)]
                    for f in range(1, OUT_CH // 16):
                        acc = acc + (sr[b][e, pl.ds(f * 16, 16)]
                                     * dr[b][e, pl.ds(f * 16, 16)])
                    vec = jnp.where(lane == j, jnp.sum(acc), vec)
                out_v[pl.ds(i * DK + g * 16, 16)] = vec

            @pl.when(io < DNB // 2 - 1)
            def _():
                pltpu.async_copy(z.at[sl_all.at[pl.ds((i + 2) * DK, DK)]],
                                 sr[b], ssem[b])
                pltpu.async_copy(z.at[dl_all.at[pl.ds((i + 2) * DK, DK)]],
                                 dr[b], dsem[b])

    pltpu.sync_copy(out_v, scores.at[pl.ds(w * DNB * DK, DNB * DK)])


_decode_kernel = pl.kernel(
    _decode_body,
    out_type=jax.ShapeDtypeStruct((L_PAD,), _f32),
    mesh=plsc.VectorSubcoreMesh(core_axis_name="c", subcore_axis_name="s"),
    compiler_params=pltpu.CompilerParams(use_tc_tiling_on_sc=False,
                                         needs_layout_passes=False),
    scratch_types=[
        pltpu.VMEM((DNB * DK,), _i32),
        pltpu.VMEM((DNB * DK,), _i32),
        pltpu.VMEM((DK, OUT_CH), _f32),
        pltpu.VMEM((DK, OUT_CH), _f32),
        pltpu.VMEM((DK, OUT_CH), _f32),
        pltpu.VMEM((DK, OUT_CH), _f32),
        pltpu.VMEM((DNB * DK,), _f32),
        pltpu.SemaphoreType.DMA,
        pltpu.SemaphoreType.DMA,
        pltpu.SemaphoreType.DMA,
        pltpu.SemaphoreType.DMA,
    ],
)


_R = 1000


def _encode_tc_body(a0, a1, cnt, xb, w1l0, w1l1, w1r, b1, w2l, w2r, b2,
                    p_out, r_out):
    inv = 1.0 / jnp.maximum(cnt[:, 0:1], 1.0)
    h = (jnp.dot(a0[...] * inv, w1l0[...], preferred_element_type=_f32)
         + jnp.dot(a1[...] * inv, w1l1[...], preferred_element_type=_f32)
         + jnp.dot(xb[...], w1r[...], preferred_element_type=_f32)
         + b1[...])
    h = jnp.maximum(h, 0.0)
    p_out[...] = jnp.dot(h, w2l[...], preferred_element_type=_f32)
    r_out[...] = jnp.dot(h, w2r[...], preferred_element_type=_f32) + b2[...]


def _combine_tc_body(a0, a1, cnt, r, z_out):
    inv = 1.0 / jnp.maximum(cnt[:, 0:1], 1.0)
    m = jnp.concatenate([a0[...], a1[...]], axis=-1)
    z_out[...] = m * inv + r[...]


def _full(shape):
    return pl.BlockSpec(shape, lambda i: (0,) * len(shape))


def _rows(shape):
    return pl.BlockSpec(shape, lambda i: (i,) + (0,) * (len(shape) - 1))


_encode_tc = pl.pallas_call(
    _encode_tc_body,
    grid=(N_NODES // _R,),
    in_specs=[
        _rows((_R, HALF)), _rows((_R, HALF)), _rows((_R, 16)),
        _rows((_R, IN_CH)),
        _full((HALF, HID_CH)), _full((HALF, HID_CH)), _full((IN_CH, HID_CH)),
        _full((1, HID_CH)),
        _full((HID_CH, OUT_CH)), _full((HID_CH, OUT_CH)), _full((1, OUT_CH)),
    ],
    out_specs=[_rows((_R, OUT_CH)), _rows((_R, OUT_CH))],
    out_shape=[
        jax.ShapeDtypeStruct((N_NODES, OUT_CH), _f32),
        jax.ShapeDtypeStruct((N_NODES, OUT_CH), _f32),
    ],
)

_combine_tc = pl.pallas_call(
    _combine_tc_body,
    grid=(N_NODES // _R,),
    in_specs=[
        _rows((_R, HALF)), _rows((_R, HALF)), _rows((_R, 16)),
        _rows((_R, OUT_CH)),
    ],
    out_specs=_rows((_R, OUT_CH)),
    out_shape=jax.ShapeDtypeStruct((N_NODES, OUT_CH), _f32),
)


def kernel(x, edge_index, edge_label_index, W1l, W1r, b1, W2l, W2r, b2):
    src = edge_index[0].astype(_i32)
    dst = edge_index[1].astype(_i32)
    sl = edge_label_index[0].astype(_i32)
    dl = edge_label_index[1].astype(_i32)

    ep = E_PAD - N_EDGES
    pad_src = (jnp.arange(ep, dtype=_i32) * 37) % N_NODES
    pad_dst = N_NODES + (jnp.arange(ep, dtype=_i32) % (N_PAD_ROWS - N_NODES))
    src_p = jnp.concatenate([src, pad_src]).reshape(NT, NB, KB)
    srcs2 = jnp.stack([2 * src_p, 2 * src_p + 1])
    dsts = jnp.concatenate([dst, pad_dst]).reshape(NT, NB, KB)

    agg1, cnt = _make_agg_kernel(True)(x.reshape(2 * N_NODES, HALF),
                                       srcs2, dsts)
    p, r = _encode_tc(agg1[0], agg1[1], cnt, x,
                      W1l[:HALF], W1l[HALF:], W1r, b1.reshape(1, HID_CH),
                      W2l, W2r, b2.reshape(1, OUT_CH))

    agg2 = _make_agg_kernel(False)(p.reshape(2 * N_NODES, HALF), srcs2, dsts)
    z = _combine_tc(agg2[0], agg2[1], cnt, r)

    lp = L_PAD - N_LABEL
    pad_l = (jnp.arange(lp, dtype=_i32) * 41) % N_NODES
    sls = jnp.concatenate([sl, pad_l]).reshape(NW, DNB * DK)
    dls = jnp.concatenate([dl, pad_l]).reshape(NW, DNB * DK)
    scores = _decode_kernel(z, sls, dls)
    return scores[:N_LABEL]
